# baseline (speedup 1.0000x reference)
import functools

import jax
import jax.numpy as jnp
from jax.experimental import pallas as pl


def _mlp_block_kernel(x_ref, w1_ref, b1_ref, w2_ref, b2_ref, o_ref):
    h = jnp.dot(x_ref[...], w1_ref[...], preferred_element_type=jnp.float32)
    h = jnp.maximum(h + b1_ref[...], 0.0)
    o = jnp.dot(h, w2_ref[...], preferred_element_type=jnp.float32)
    o_ref[...] = o + b2_ref[...]


@functools.partial(jax.jit, static_argnums=(5,))
def _mlp_pallas(x, w1, b1, w2, b2, bm):
    m, i = x.shape
    h = w1.shape[1]
    o = w2.shape[1]
    mp = ((m + bm - 1) // bm) * bm
    if mp != m:
        x = jnp.pad(x, ((0, mp - m), (0, 0)))
    out = pl.pallas_call(
        _mlp_block_kernel,
        grid=(mp // bm,),
        in_specs=[
            pl.BlockSpec((bm, i), lambda g: (g, 0)),
            pl.BlockSpec((i, h), lambda g: (0, 0)),
            pl.BlockSpec((1, h), lambda g: (0, 0)),
            pl.BlockSpec((h, o), lambda g: (0, 0)),
            pl.BlockSpec((1, o), lambda g: (0, 0)),
        ],
        out_specs=pl.BlockSpec((bm, o), lambda g: (g, 0)),
        out_shape=jax.ShapeDtypeStruct((mp, o), jnp.float32),
    )(x, w1, b1.reshape(1, h), w2, b2.reshape(1, o))
    return out[:m]


def _mlp(p, name, x, bm):
    return _mlp_pallas(x, p[name + "_W1"], p[name + "_b1"],
                       p[name + "_W2"], p[name + "_b2"], bm)


def _mlp_parts_kernel(nparts, *refs):
    part_refs = refs[:nparts]
    w1_refs = refs[nparts:2 * nparts]
    b1_ref, w2_ref, b2_ref, o_ref = refs[2 * nparts:]
    h = b1_ref[...]
    for pr, wr in zip(part_refs, w1_refs):
        h = h + jnp.dot(pr[...], wr[...], preferred_element_type=jnp.float32)
    h = jnp.maximum(h, 0.0)
    o = jnp.dot(h, w2_ref[...], preferred_element_type=jnp.float32)
    o_ref[...] = o + b2_ref[...]


def _mlp_parts(p, name, parts, bm):
    import functools as _ft
    w1, b1 = p[name + "_W1"], p[name + "_b1"]
    w2, b2 = p[name + "_W2"], p[name + "_b2"]
    m = parts[0].shape[0]
    h = w1.shape[1]
    o = w2.shape[1]
    mp = ((m + bm - 1) // bm) * bm
    if mp != m:
        parts = [jnp.pad(a, ((0, mp - m), (0, 0))) for a in parts]
    w1_chunks = []
    off = 0
    for a in parts:
        w1_chunks.append(w1[off:off + a.shape[1]])
        off += a.shape[1]
    in_specs = (
        [pl.BlockSpec((bm, a.shape[1]), lambda g: (g, 0)) for a in parts]
        + [pl.BlockSpec(w.shape, lambda g: (0, 0)) for w in w1_chunks]
        + [pl.BlockSpec((1, h), lambda g: (0, 0)),
           pl.BlockSpec((h, o), lambda g: (0, 0)),
           pl.BlockSpec((1, o), lambda g: (0, 0))]
    )
    out = pl.pallas_call(
        _ft.partial(_mlp_parts_kernel, len(parts)),
        grid=(mp // bm,),
        in_specs=in_specs,
        out_specs=pl.BlockSpec((bm, o), lambda g: (g, 0)),
        out_shape=jax.ShapeDtypeStruct((mp, o), jnp.float32),
    )(*parts, *w1_chunks, b1.reshape(1, h), w2, b2.reshape(1, o))
    return out[:m]


def _scatter_mean(data, ids, num_segments):
    s = jax.ops.segment_sum(data, ids, num_segments=num_segments)
    c = jax.ops.segment_sum(jnp.ones((data.shape[0], 1), data.dtype), ids,
                            num_segments=num_segments)
    return s / jnp.clip(c, 1.0, None)


def _meta(x, edge_index, edge_attr, u, batch, params, i):
    row = edge_index[0]
    col = edge_index[1]
    eb = batch[row]
    edge_attr = _mlp_parts(params, "e%d" % i,
                           [x[row], x[col], edge_attr, u[eb]], 4096)
    agg = _scatter_mean(edge_attr, row, x.shape[0])
    x = _mlp_parts(params, "n%d" % i, [x, agg, u[batch]], 4096)
    g_in = jnp.concatenate([u,
                            _scatter_mean(x, batch, u.shape[0]),
                            _scatter_mean(edge_attr, eb, u.shape[0])], axis=1)
    u = _mlp(params, "g%d" % i, g_in, 64)
    return x, edge_attr, u


def kernel(x, edge_index, edge_weight, u, batch, params):
    x, e, u = _meta(x, edge_index, edge_weight, u, batch, params, 1)
    x, e, u = _meta(x, edge_index, e, u, batch, params, 2)
    x, e, u = _meta(x, edge_index, e, u, batch, params, 3)
    return jax.nn.sigmoid(u)
